# Initial kernel scaffold; baseline (speedup 1.0000x reference)
import jax
import jax.numpy as jnp
from jax.experimental import pallas as pl


def kernel(pos1, pos2, feature1, feature2, W0, gamma0, beta0, W1, gamma1, beta1, W2, gamma2, beta2):
    raise NotImplementedError("write your pallas kernel here")


# speedup vs baseline: 11.2655x; 11.2655x over previous
import functools

import jax
import jax.numpy as jnp
from jax import lax
from jax.experimental import pallas as pl
from jax.experimental.pallas import tpu as pltpu
from jax.experimental.pallas import tpu_sc as plsc

B, N, C = 8, 2048, 64
S = 16
RADIUS2 = 100.0
EPS = 1e-5
M = B * N * S
R = B * N * S

TQ = 256
TR = 2048

_F32 = jnp.float32


def _pre_body(p1t, p2t, f1t, f2t, wpt, wf2t, wf1t, g2_ref, q_ref):
    g2_ref[0] = (
        jnp.dot(p2t[0], wpt[...], preferred_element_type=_F32)
        + jnp.dot(f2t[0], wf2t[...], preferred_element_type=_F32)
    )
    q_ref[0] = (
        jnp.dot(f1t[0], wf1t[...], preferred_element_type=_F32)
        - jnp.dot(p1t[0], wpt[...], preferred_element_type=_F32)
    )


def _pre(p1t, p2t, f1t, f2t, wpt, wf2t, wf1t):
    bs_pt = pl.BlockSpec((1, N, 3), lambda b: (b, 0, 0))
    bs_ft = pl.BlockSpec((1, N, C), lambda b: (b, 0, 0))
    bs_w3 = pl.BlockSpec((3, C), lambda b: (0, 0))
    bs_wc = pl.BlockSpec((C, C), lambda b: (0, 0))
    return pl.pallas_call(
        _pre_body,
        grid=(B,),
        in_specs=[bs_pt, bs_pt, bs_ft, bs_ft, bs_w3, bs_wc, bs_wc],
        out_specs=[
            pl.BlockSpec((1, N, C), lambda b: (b, 0, 0)),
            pl.BlockSpec((1, N, C), lambda b: (b, 0, 0)),
        ],
        out_shape=[
            jax.ShapeDtypeStruct((B, N, C), _F32),
            jax.ShapeDtypeStruct((B, N, C), _F32),
        ],
    )(p1t, p2t, f1t, f2t, wpt, wf2t, wf1t)


def _knn_body(p1t_ref, p2_ref, idx_ref):
    b = pl.program_id(0)
    x1 = p1t_ref[0][:, 0:1]
    y1 = p1t_ref[0][:, 1:2]
    z1 = p1t_ref[0][:, 2:3]
    x2 = p2_ref[0][0:1, :]
    y2 = p2_ref[0][1:2, :]
    z2 = p2_ref[0][2:3, :]
    dx = x1 - x2
    dy = y1 - y2
    dz = z1 - z2
    d2 = dx * dx + dy * dy + dz * dz

    iota = lax.broadcasted_iota(jnp.int32, (TQ, N), 1)
    cols = []
    a0 = None
    for k in range(S):
        m = jnp.min(d2, axis=1, keepdims=True)
        cand = jnp.where(d2 == m, iota, jnp.int32(N))
        a = jnp.min(cand, axis=1, keepdims=True)
        if k == 0:
            a0 = a
        cols.append(jnp.where(m > RADIUS2, a0, a))
        d2 = jnp.where(iota == a, jnp.inf, d2)
    idx_ref[0] = jnp.concatenate(cols, axis=1) + b * N


def _knn(p1t, pos2):
    return pl.pallas_call(
        _knn_body,
        grid=(B, N // TQ),
        in_specs=[
            pl.BlockSpec((1, TQ, 3), lambda b, j: (b, j, 0)),
            pl.BlockSpec((1, 3, N), lambda b, j: (b, 0, 0)),
        ],
        out_specs=pl.BlockSpec((1, TQ, S), lambda b, j: (b, j, 0)),
        out_shape=jax.ShapeDtypeStruct((B, N, S), jnp.int32),
    )(p1t, pos2)


_NW = 32
_PER_W = R // _NW
_CH = 128
_GRP = 8
_NGRP = _PER_W // (_CH * _GRP)


def _sc_gather(g2flat, idx2d):
    mesh = plsc.VectorSubcoreMesh(core_axis_name="c", subcore_axis_name="s")

    @functools.partial(
        pl.kernel,
        mesh=mesh,
        compiler_params=pltpu.CompilerParams(use_tc_tiling_on_sc=False),
        out_type=jax.ShapeDtypeStruct((R, C), _F32),
        scratch_types=[
            pltpu.VMEM((_PER_W // _CH, _CH), jnp.int32),
            pltpu.VMEM((_GRP * _CH, C), _F32),
            pltpu.SemaphoreType.DMA,
        ],
    )
    def k(tab_hbm, idx_hbm, out_hbm, idx_v, rows_v, sem):
        wid = lax.axis_index("s") * 2 + lax.axis_index("c")
        base = wid * _PER_W
        pltpu.sync_copy(idx_hbm.at[pl.ds(wid * (_PER_W // _CH), _PER_W // _CH)], idx_v)
        for g in range(_NGRP):
            copies = []
            for j in range(_GRP):
                copies.append(
                    pltpu.async_copy(
                        tab_hbm.at[idx_v.at[g * _GRP + j]],
                        rows_v.at[pl.ds(j * _CH, _CH)],
                        sem,
                    )
                )
            for cp in copies:
                cp.wait()
            pltpu.sync_copy(
                rows_v, out_hbm.at[pl.ds(base + g * _GRP * _CH, _GRP * _CH)]
            )

    return k(g2flat, idx2d)


def _pack_stats(y, ch):
    s1 = jnp.sum(y, axis=0).reshape(1, ch)
    s2 = jnp.sum(y * y, axis=0).reshape(1, ch)
    return jnp.concatenate([s1, s2, jnp.zeros((6, ch), _F32)], axis=0)


def _bn_coeffs(stats, gb):
    mean = stats[0:1, :] * (1.0 / M)
    var = stats[1:2, :] * (1.0 / M) - mean * mean
    inv = lax.rsqrt(var + EPS)
    scale = inv * gb[0:1, :]
    shift = gb[1:2, :] - mean * scale
    return scale, shift


def _stats1_body(g_ref, q_ref, st_ref):
    y = g_ref[...].reshape(TR // S, S, C) + q_ref[...][:, None, :]
    y = y.reshape(TR, C)
    part = _pack_stats(y, C)

    @pl.when(pl.program_id(0) == 0)
    def _():
        st_ref[...] = jnp.zeros_like(st_ref)

    st_ref[...] += part


def _stats1(G, qflat):
    return pl.pallas_call(
        _stats1_body,
        grid=(R // TR,),
        in_specs=[
            pl.BlockSpec((TR, C), lambda i: (i, 0)),
            pl.BlockSpec((TR // S, C), lambda i: (i, 0)),
        ],
        out_specs=pl.BlockSpec((8, C), lambda i: (0, 0)),
        out_shape=jax.ShapeDtypeStruct((8, C), _F32),
    )(G, qflat)


def _layer1_body(g_ref, q_ref, st1_ref, gb_ref, w1t_ref, y2_ref, st2_ref):
    scale, shift = _bn_coeffs(st1_ref[...], gb_ref[...])
    y1 = g_ref[...].reshape(TR // S, S, C) + q_ref[...][:, None, :]
    y1 = y1.reshape(TR, C)
    h = jnp.maximum(y1 * scale + shift, 0.0)
    y2 = jnp.dot(h, w1t_ref[...], preferred_element_type=_F32)
    y2_ref[...] = y2

    @pl.when(pl.program_id(0) == 0)
    def _():
        st2_ref[...] = jnp.zeros_like(st2_ref)

    st2_ref[...] += _pack_stats(y2, C)


def _layer1(G, qflat, st1, gb0, w1t):
    return pl.pallas_call(
        _layer1_body,
        grid=(R // TR,),
        in_specs=[
            pl.BlockSpec((TR, C), lambda i: (i, 0)),
            pl.BlockSpec((TR // S, C), lambda i: (i, 0)),
            pl.BlockSpec((8, C), lambda i: (0, 0)),
            pl.BlockSpec((8, C), lambda i: (0, 0)),
            pl.BlockSpec((C, C), lambda i: (0, 0)),
        ],
        out_specs=[
            pl.BlockSpec((TR, C), lambda i: (i, 0)),
            pl.BlockSpec((8, C), lambda i: (0, 0)),
        ],
        out_shape=[
            jax.ShapeDtypeStruct((R, C), _F32),
            jax.ShapeDtypeStruct((8, C), _F32),
        ],
    )(G, qflat, st1, gb0, w1t)


C3 = 128


def _layer2_body(y2_ref, st2_ref, gb_ref, w2t_ref, ym_ref, st3_ref):
    scale, shift = _bn_coeffs(st2_ref[...], gb_ref[...])
    h = jnp.maximum(y2_ref[...] * scale + shift, 0.0)
    y3 = jnp.dot(h, w2t_ref[...], preferred_element_type=_F32)
    ym_ref[...] = jnp.max(y3.reshape(TR // S, S, C3), axis=1)

    @pl.when(pl.program_id(0) == 0)
    def _():
        st3_ref[...] = jnp.zeros_like(st3_ref)

    st3_ref[...] += _pack_stats(y3, C3)


def _layer2(y2, st2, gb1, w2t):
    return pl.pallas_call(
        _layer2_body,
        grid=(R // TR,),
        in_specs=[
            pl.BlockSpec((TR, C), lambda i: (i, 0)),
            pl.BlockSpec((8, C), lambda i: (0, 0)),
            pl.BlockSpec((8, C), lambda i: (0, 0)),
            pl.BlockSpec((C, C3), lambda i: (0, 0)),
        ],
        out_specs=[
            pl.BlockSpec((TR // S, C3), lambda i: (i, 0)),
            pl.BlockSpec((8, C3), lambda i: (0, 0)),
        ],
        out_shape=[
            jax.ShapeDtypeStruct((B * N, C3), _F32),
            jax.ShapeDtypeStruct((8, C3), _F32),
        ],
    )(y2, st2, gb1, w2t)


TF = 512


def _final_body(ym_ref, st3_ref, gb_ref, out_ref):
    scale, shift = _bn_coeffs(st3_ref[...], gb_ref[...])
    t = jnp.maximum(ym_ref[0] * scale + shift, 0.0)
    out_ref[0] = t.T


def _final(ym3, st3, gb2):
    return pl.pallas_call(
        _final_body,
        grid=(B, N // TF),
        in_specs=[
            pl.BlockSpec((1, TF, C3), lambda b, j: (b, j, 0)),
            pl.BlockSpec((8, C3), lambda b, j: (0, 0)),
            pl.BlockSpec((8, C3), lambda b, j: (0, 0)),
        ],
        out_specs=pl.BlockSpec((1, C3, TF), lambda b, j: (b, 0, j)),
        out_shape=jax.ShapeDtypeStruct((B, C3, N), _F32),
    )(ym3.reshape(B, N, C3), st3, gb2)


def kernel(pos1, pos2, feature1, feature2, W0, gamma0, beta0, W1, gamma1,
           beta1, W2, gamma2, beta2):
    p1t = jnp.transpose(pos1, (0, 2, 1))
    p2t = jnp.transpose(pos2, (0, 2, 1))
    f1t = jnp.transpose(feature1, (0, 2, 1))
    f2t = jnp.transpose(feature2, (0, 2, 1))

    wpt = jnp.transpose(W0[:, :3])
    wf2t = jnp.transpose(W0[:, 3:3 + C])
    wf1t = jnp.transpose(W0[:, 3 + C:])
    w1t = jnp.transpose(W1)
    w2t = jnp.transpose(W2)

    def gb(g, b, ch):
        z = jnp.zeros((8, ch), _F32)
        return z.at[0, :].set(g).at[1, :].set(b)

    gb0 = gb(gamma0, beta0, C)
    gb1 = gb(gamma1, beta1, C)
    gb2 = gb(gamma2, beta2, C3)

    g2, q = _pre(p1t, p2t, f1t, f2t, wpt, wf2t, wf1t)
    idx = _knn(p1t, pos2)
    G = _sc_gather(g2.reshape(B * N, C), idx.reshape(R // _CH, _CH))
    qflat = q.reshape(B * N, C)
    st1 = _stats1(G, qflat)
    y2, st2 = _layer1(G, qflat, st1, gb0, w1t)
    ym3, st3 = _layer2(y2, st2, gb1, w2t)
    feat1_new = _final(ym3, st3, gb2)
    return (pos1, feat1_new)


# speedup vs baseline: 15.9048x; 1.4118x over previous
import functools

import jax
import jax.numpy as jnp
from jax import lax
from jax.experimental import pallas as pl
from jax.experimental.pallas import tpu as pltpu
from jax.experimental.pallas import tpu_sc as plsc

B, N, C = 8, 2048, 64
S = 16
RADIUS2 = 100.0
EPS = 1e-5
M = B * N * S
R = B * N * S

TQ = 512
TR = 4096

_F32 = jnp.float32


def _pre_body(p1t, p2t, f1t, f2t, wpt, wf2t, wf1t, g2_ref, q_ref):
    g2_ref[0] = (
        jnp.dot(p2t[0], wpt[...], preferred_element_type=_F32)
        + jnp.dot(f2t[0], wf2t[...], preferred_element_type=_F32)
    )
    q_ref[0] = (
        jnp.dot(f1t[0], wf1t[...], preferred_element_type=_F32)
        - jnp.dot(p1t[0], wpt[...], preferred_element_type=_F32)
    )


def _pre(p1t, p2t, f1t, f2t, wpt, wf2t, wf1t):
    bs_pt = pl.BlockSpec((1, N, 3), lambda b: (b, 0, 0))
    bs_ft = pl.BlockSpec((1, N, C), lambda b: (b, 0, 0))
    bs_w3 = pl.BlockSpec((3, C), lambda b: (0, 0))
    bs_wc = pl.BlockSpec((C, C), lambda b: (0, 0))
    return pl.pallas_call(
        _pre_body,
        grid=(B,),
        in_specs=[bs_pt, bs_pt, bs_ft, bs_ft, bs_w3, bs_wc, bs_wc],
        out_specs=[
            pl.BlockSpec((1, N, C), lambda b: (b, 0, 0)),
            pl.BlockSpec((1, N, C), lambda b: (b, 0, 0)),
        ],
        out_shape=[
            jax.ShapeDtypeStruct((B, N, C), _F32),
            jax.ShapeDtypeStruct((B, N, C), _F32),
        ],
    )(p1t, p2t, f1t, f2t, wpt, wf2t, wf1t)


def _knn_body(p1t_ref, p2_ref, idx_ref):
    b = pl.program_id(0)
    x1 = p1t_ref[0][:, 0:1]
    y1 = p1t_ref[0][:, 1:2]
    z1 = p1t_ref[0][:, 2:3]
    x2 = p2_ref[0][0:1, :]
    y2 = p2_ref[0][1:2, :]
    z2 = p2_ref[0][2:3, :]
    dx = x1 - x2
    dy = y1 - y2
    dz = z1 - z2
    d2 = dx * dx + dy * dy + dz * dz

    iota = lax.broadcasted_iota(jnp.int32, (TQ, N), 1)
    packed = (lax.bitcast_convert_type(d2, jnp.int32) & jnp.int32(~0x7FF)) | iota
    big = jnp.int32(0x7F800000)
    cols = []
    m0 = None
    for k in range(S):
        m = jnp.min(packed, axis=1, keepdims=True)
        if k == 0:
            m0 = m
        cols.append(m)
        packed = jnp.where(packed == m, big, packed)
    mk = jnp.concatenate(cols, axis=1)
    m0 = jnp.broadcast_to(m0, (TQ, S))
    d2k = lax.bitcast_convert_type(mk & jnp.int32(~0x7FF), _F32)
    sel = jnp.where(d2k > RADIUS2, m0, mk)
    idx_ref[0] = (sel & jnp.int32(0x7FF)) + b * N


def _knn(p1t, pos2):
    return pl.pallas_call(
        _knn_body,
        grid=(B, N // TQ),
        in_specs=[
            pl.BlockSpec((1, TQ, 3), lambda b, j: (b, j, 0)),
            pl.BlockSpec((1, 3, N), lambda b, j: (b, 0, 0)),
        ],
        out_specs=pl.BlockSpec((1, TQ, S), lambda b, j: (b, j, 0)),
        out_shape=jax.ShapeDtypeStruct((B, N, S), jnp.int32),
    )(p1t, pos2)


_NW = 32
_PER_W = R // _NW
_CH = 128
_GRP = 8
_NGRP = _PER_W // (_CH * _GRP)


def _sc_gather(g2flat, idx2d):
    mesh = plsc.VectorSubcoreMesh(core_axis_name="c", subcore_axis_name="s")

    @functools.partial(
        pl.kernel,
        mesh=mesh,
        compiler_params=pltpu.CompilerParams(use_tc_tiling_on_sc=False),
        out_type=jax.ShapeDtypeStruct((R, C), _F32),
        scratch_types=[
            pltpu.VMEM((_PER_W // _CH, _CH), jnp.int32),
            pltpu.VMEM((_GRP * _CH, C), _F32),
            pltpu.SemaphoreType.DMA,
        ],
    )
    def k(tab_hbm, idx_hbm, out_hbm, idx_v, rows_v, sem):
        wid = lax.axis_index("s") * 2 + lax.axis_index("c")
        base = wid * _PER_W
        pltpu.sync_copy(idx_hbm.at[pl.ds(wid * (_PER_W // _CH), _PER_W // _CH)], idx_v)
        for g in range(_NGRP):
            copies = []
            for j in range(_GRP):
                copies.append(
                    pltpu.async_copy(
                        tab_hbm.at[idx_v.at[g * _GRP + j]],
                        rows_v.at[pl.ds(j * _CH, _CH)],
                        sem,
                    )
                )
            for cp in copies:
                cp.wait()
            pltpu.sync_copy(
                rows_v, out_hbm.at[pl.ds(base + g * _GRP * _CH, _GRP * _CH)]
            )

    return k(g2flat, idx2d)


def _pack_stats(y, ch):
    s1 = jnp.sum(y, axis=0).reshape(1, ch)
    s2 = jnp.sum(y * y, axis=0).reshape(1, ch)
    return jnp.concatenate([s1, s2, jnp.zeros((6, ch), _F32)], axis=0)


def _bn_coeffs(stats, gb):
    mean = stats[0:1, :] * (1.0 / M)
    var = stats[1:2, :] * (1.0 / M) - mean * mean
    inv = lax.rsqrt(var + EPS)
    scale = inv * gb[0:1, :]
    shift = gb[1:2, :] - mean * scale
    return scale, shift


def _stats1_body(g_ref, q_ref, st_ref):
    y = g_ref[...].reshape(TR // S, S, C) + q_ref[...][:, None, :]
    y = y.reshape(TR, C)
    part = _pack_stats(y, C)

    @pl.when(pl.program_id(0) == 0)
    def _():
        st_ref[...] = jnp.zeros_like(st_ref)

    st_ref[...] += part


def _stats1(G, qflat):
    return pl.pallas_call(
        _stats1_body,
        grid=(R // TR,),
        in_specs=[
            pl.BlockSpec((TR, C), lambda i: (i, 0)),
            pl.BlockSpec((TR // S, C), lambda i: (i, 0)),
        ],
        out_specs=pl.BlockSpec((8, C), lambda i: (0, 0)),
        out_shape=jax.ShapeDtypeStruct((8, C), _F32),
    )(G, qflat)


def _layer1_body(g_ref, q_ref, st1_ref, gb_ref, w1t_ref, y2_ref, st2_ref):
    scale, shift = _bn_coeffs(st1_ref[...], gb_ref[...])
    y1 = g_ref[...].reshape(TR // S, S, C) + q_ref[...][:, None, :]
    y1 = y1.reshape(TR, C)
    h = jnp.maximum(y1 * scale + shift, 0.0)
    y2 = jnp.dot(h, w1t_ref[...], preferred_element_type=_F32)
    y2_ref[...] = y2

    @pl.when(pl.program_id(0) == 0)
    def _():
        st2_ref[...] = jnp.zeros_like(st2_ref)

    st2_ref[...] += _pack_stats(y2, C)


def _layer1(G, qflat, st1, gb0, w1t):
    return pl.pallas_call(
        _layer1_body,
        grid=(R // TR,),
        in_specs=[
            pl.BlockSpec((TR, C), lambda i: (i, 0)),
            pl.BlockSpec((TR // S, C), lambda i: (i, 0)),
            pl.BlockSpec((8, C), lambda i: (0, 0)),
            pl.BlockSpec((8, C), lambda i: (0, 0)),
            pl.BlockSpec((C, C), lambda i: (0, 0)),
        ],
        out_specs=[
            pl.BlockSpec((TR, C), lambda i: (i, 0)),
            pl.BlockSpec((8, C), lambda i: (0, 0)),
        ],
        out_shape=[
            jax.ShapeDtypeStruct((R, C), _F32),
            jax.ShapeDtypeStruct((8, C), _F32),
        ],
    )(G, qflat, st1, gb0, w1t)


C3 = 128


def _layer2_body(y2_ref, st2_ref, gb_ref, w2t_ref, ym_ref, st3_ref):
    scale, shift = _bn_coeffs(st2_ref[...], gb_ref[...])
    h = jnp.maximum(y2_ref[...] * scale + shift, 0.0)
    y3 = jnp.dot(h, w2t_ref[...], preferred_element_type=_F32)
    ym_ref[...] = jnp.max(y3.reshape(TR // S, S, C3), axis=1)

    @pl.when(pl.program_id(0) == 0)
    def _():
        st3_ref[...] = jnp.zeros_like(st3_ref)

    st3_ref[...] += _pack_stats(y3, C3)


def _layer2(y2, st2, gb1, w2t):
    return pl.pallas_call(
        _layer2_body,
        grid=(R // TR,),
        in_specs=[
            pl.BlockSpec((TR, C), lambda i: (i, 0)),
            pl.BlockSpec((8, C), lambda i: (0, 0)),
            pl.BlockSpec((8, C), lambda i: (0, 0)),
            pl.BlockSpec((C, C3), lambda i: (0, 0)),
        ],
        out_specs=[
            pl.BlockSpec((TR // S, C3), lambda i: (i, 0)),
            pl.BlockSpec((8, C3), lambda i: (0, 0)),
        ],
        out_shape=[
            jax.ShapeDtypeStruct((B * N, C3), _F32),
            jax.ShapeDtypeStruct((8, C3), _F32),
        ],
    )(y2, st2, gb1, w2t)


TF = 512


def _final_body(ym_ref, st3_ref, gb_ref, out_ref):
    scale, shift = _bn_coeffs(st3_ref[...], gb_ref[...])
    t = jnp.maximum(ym_ref[0] * scale + shift, 0.0)
    out_ref[0] = t.T


def _final(ym3, st3, gb2):
    return pl.pallas_call(
        _final_body,
        grid=(B, N // TF),
        in_specs=[
            pl.BlockSpec((1, TF, C3), lambda b, j: (b, j, 0)),
            pl.BlockSpec((8, C3), lambda b, j: (0, 0)),
            pl.BlockSpec((8, C3), lambda b, j: (0, 0)),
        ],
        out_specs=pl.BlockSpec((1, C3, TF), lambda b, j: (b, 0, j)),
        out_shape=jax.ShapeDtypeStruct((B, C3, N), _F32),
    )(ym3.reshape(B, N, C3), st3, gb2)


def kernel(pos1, pos2, feature1, feature2, W0, gamma0, beta0, W1, gamma1,
           beta1, W2, gamma2, beta2):
    p1t = jnp.transpose(pos1, (0, 2, 1))
    p2t = jnp.transpose(pos2, (0, 2, 1))
    f1t = jnp.transpose(feature1, (0, 2, 1))
    f2t = jnp.transpose(feature2, (0, 2, 1))

    wpt = jnp.transpose(W0[:, :3])
    wf2t = jnp.transpose(W0[:, 3:3 + C])
    wf1t = jnp.transpose(W0[:, 3 + C:])
    w1t = jnp.transpose(W1)
    w2t = jnp.transpose(W2)

    def gb(g, b, ch):
        z = jnp.zeros((8, ch), _F32)
        return z.at[0, :].set(g).at[1, :].set(b)

    gb0 = gb(gamma0, beta0, C)
    gb1 = gb(gamma1, beta1, C)
    gb2 = gb(gamma2, beta2, C3)

    g2, q = _pre(p1t, p2t, f1t, f2t, wpt, wf2t, wf1t)
    idx = _knn(p1t, pos2)
    G = _sc_gather(g2.reshape(B * N, C), idx.reshape(R // _CH, _CH))
    qflat = q.reshape(B * N, C)
    st1 = _stats1(G, qflat)
    y2, st2 = _layer1(G, qflat, st1, gb0, w1t)
    ym3, st3 = _layer2(y2, st2, gb1, w2t)
    feat1_new = _final(ym3, st3, gb2)
    return (pos1, feat1_new)
